# Initial kernel scaffold; baseline (speedup 1.0000x reference)
import functools

import jax
import jax.numpy as jnp
from jax import lax
from jax.experimental import pallas as pl
from jax.experimental.pallas import tpu as pltpu
from jax.experimental.pallas import tpu_sc as plsc

N_NODES = 10000
N_EDGES = 320000
FDIM = 128

_info = plsc.get_sparse_core_info()
NUM_CORES = _info.num_cores
NUM_SUBCORES = _info.num_subcores
LANES = _info.num_lanes

NUM_TILES = NUM_CORES * NUM_SUBCORES
EDGES_PER_TILE = N_EDGES // NUM_TILES
CHUNK = 80
N_CHUNKS = EDGES_PER_TILE // CHUNK
ROWS_PER_TILE = N_NODES // NUM_SUBCORES
ZROWS = 125


def _sc_aggregate(support, src, dst, w):
    mesh = plsc.VectorSubcoreMesh(core_axis_name="c", subcore_axis_name="s")

    @functools.partial(
        pl.kernel,
        out_type=jax.ShapeDtypeStruct((NUM_CORES, N_NODES, FDIM), jnp.float32),
        mesh=mesh,
        scratch_types=[
            pltpu.VMEM_SHARED((N_NODES, FDIM), jnp.float32),
            pltpu.VMEM((CHUNK,), jnp.int32),
            pltpu.VMEM((CHUNK,), jnp.int32),
            pltpu.VMEM((CHUNK,), jnp.float32),
            pltpu.VMEM((CHUNK, FDIM), jnp.float32),
            pltpu.VMEM((ZROWS, FDIM), jnp.float32),
            pltpu.SemaphoreType.DMA,
        ],
    )
    def k(support_hbm, src_hbm, dst_hbm, w_hbm, out_hbm,
          acc, srcbuf, dstbuf, wbuf, rowbuf, zbuf, sem):
        c = lax.axis_index("c")
        s = lax.axis_index("s")
        base = (c * NUM_SUBCORES + s) * EDGES_PER_TILE
        row0 = s * ROWS_PER_TILE

        zeros = jnp.zeros((LANES,), jnp.float32)

        def zrow(i, carry):
            for j in range(FDIM // LANES):
                zbuf[i, pl.ds(j * LANES, LANES)] = zeros
            return carry

        lax.fori_loop(0, ZROWS, zrow, 0)
        for z in range(ROWS_PER_TILE // ZROWS):
            pltpu.sync_copy(zbuf, acc.at[pl.ds(row0 + z * ZROWS, ZROWS)])
        plsc.subcore_barrier()

        def chunk_body(ci, carry):
            off = base + ci * CHUNK
            pltpu.sync_copy(src_hbm.at[pl.ds(off, CHUNK)], srcbuf)
            g = pltpu.async_copy(support_hbm.at[srcbuf], rowbuf, sem)
            pltpu.sync_copy(dst_hbm.at[pl.ds(off, CHUNK)], dstbuf)
            pltpu.sync_copy(w_hbm.at[pl.ds(off, CHUNK)], wbuf)
            g.wait()

            def scale_row(r, rcarry):
                wv = wbuf[r]
                for j in range(FDIM // LANES):
                    sl = pl.ds(j * LANES, LANES)
                    rowbuf[r, sl] = rowbuf[r, sl] * wv
                return rcarry

            lax.fori_loop(0, CHUNK, scale_row, 0)
            pltpu.sync_copy(rowbuf, acc.at[dstbuf], add=True)
            return carry

        lax.fori_loop(0, N_CHUNKS, chunk_body, 0)
        plsc.subcore_barrier()
        pltpu.sync_copy(acc.at[pl.ds(row0, ROWS_PER_TILE)],
                        out_hbm.at[c, pl.ds(row0, ROWS_PER_TILE)])

    return k(support, src, dst, w)


_BLK = 2000


def _tc_mm(h, W):
    def body(h_ref, w_ref, o_ref):
        o_ref[...] = jnp.dot(h_ref[...], w_ref[...],
                             preferred_element_type=jnp.float32)

    return pl.pallas_call(
        body,
        grid=(N_NODES // _BLK,),
        in_specs=[
            pl.BlockSpec((_BLK, FDIM), lambda i: (i, 0)),
            pl.BlockSpec((FDIM, FDIM), lambda i: (0, 0)),
        ],
        out_specs=pl.BlockSpec((_BLK, FDIM), lambda i: (i, 0)),
        out_shape=jax.ShapeDtypeStruct((N_NODES, FDIM), jnp.float32),
    )(h, W)


def _tc_combine_relu_mm(p, b, W):
    def body(p_ref, b_ref, w_ref, o_ref):
        h = jnp.maximum(p_ref[0] + p_ref[1] + b_ref[...], 0.0)
        o_ref[...] = jnp.dot(h, w_ref[...], preferred_element_type=jnp.float32)

    return pl.pallas_call(
        body,
        grid=(N_NODES // _BLK,),
        in_specs=[
            pl.BlockSpec((NUM_CORES, _BLK, FDIM), lambda i: (0, i, 0)),
            pl.BlockSpec((1, FDIM), lambda i: (0, 0)),
            pl.BlockSpec((FDIM, FDIM), lambda i: (0, 0)),
        ],
        out_specs=pl.BlockSpec((_BLK, FDIM), lambda i: (i, 0)),
        out_shape=jax.ShapeDtypeStruct((N_NODES, FDIM), jnp.float32),
    )(p, b, W)


def _tc_final(p, b3, Wp1, bp1, Wp2, bp2):
    def body(p_ref, b3_ref, wp1_ref, bp1_ref, wp2_ref, bp2_ref,
             z_ref, emb_ref):
        emb = p_ref[0] + p_ref[1] + b3_ref[...]
        emb_ref[...] = emb
        h = jnp.maximum(
            jnp.dot(emb, wp1_ref[...], preferred_element_type=jnp.float32)
            + bp1_ref[...], 0.0)
        z_ref[...] = (
            jnp.dot(h, wp2_ref[...], preferred_element_type=jnp.float32)
            + bp2_ref[...])

    return pl.pallas_call(
        body,
        grid=(N_NODES // _BLK,),
        in_specs=[
            pl.BlockSpec((NUM_CORES, _BLK, FDIM), lambda i: (0, i, 0)),
            pl.BlockSpec((1, FDIM), lambda i: (0, 0)),
            pl.BlockSpec((FDIM, FDIM), lambda i: (0, 0)),
            pl.BlockSpec((1, FDIM), lambda i: (0, 0)),
            pl.BlockSpec((FDIM, FDIM), lambda i: (0, 0)),
            pl.BlockSpec((1, FDIM), lambda i: (0, 0)),
        ],
        out_specs=[
            pl.BlockSpec((_BLK, FDIM), lambda i: (i, 0)),
            pl.BlockSpec((_BLK, FDIM), lambda i: (i, 0)),
        ],
        out_shape=[
            jax.ShapeDtypeStruct((N_NODES, FDIM), jnp.float32),
            jax.ShapeDtypeStruct((N_NODES, FDIM), jnp.float32),
        ],
    )(p, b3, Wp1, bp1, Wp2, bp2)


def kernel(x, edge_index, edge_weight, W1, b1, W2, b2, W3, b3,
           Wp1, bp1, Wp2, bp2):
    src = edge_index[0]
    dst = edge_index[1]
    b1r = b1.reshape(1, FDIM)
    b2r = b2.reshape(1, FDIM)
    b3r = b3.reshape(1, FDIM)
    bp1r = bp1.reshape(1, FDIM)
    bp2r = bp2.reshape(1, FDIM)

    s1 = _tc_mm(x, W1)
    p1 = _sc_aggregate(s1, src, dst, edge_weight)
    s2 = _tc_combine_relu_mm(p1, b1r, W2)
    p2 = _sc_aggregate(s2, src, dst, edge_weight)
    s3 = _tc_combine_relu_mm(p2, b2r, W3)
    p3 = _sc_aggregate(s3, src, dst, edge_weight)
    z, emb = _tc_final(p3, b3r, Wp1, bp1, Wp2, bp2)
    return (z, emb)


# speedup vs baseline: 5.0201x; 5.0201x over previous
import functools

import jax
import jax.numpy as jnp
from jax import lax
from jax.experimental import pallas as pl
from jax.experimental.pallas import tpu as pltpu
from jax.experimental.pallas import tpu_sc as plsc

N_NODES = 10000
N_EDGES = 320000
FDIM = 128

_info = plsc.get_sparse_core_info()
NUM_CORES = _info.num_cores
NUM_SUBCORES = _info.num_subcores
LANES = _info.num_lanes

NUM_TILES = NUM_CORES * NUM_SUBCORES
EDGES_PER_TILE = N_EDGES // NUM_TILES
CHUNK = 80
N_CHUNKS = EDGES_PER_TILE // CHUNK
ROWS_PER_TILE = 624
ROWS_EXTRA_BASE = NUM_SUBCORES * ROWS_PER_TILE
ROWS_EXTRA = N_NODES - ROWS_EXTRA_BASE
ZROWS = 208


def _sc_aggregate(support, src, dst, w):
    mesh = plsc.VectorSubcoreMesh(core_axis_name="c", subcore_axis_name="s")

    @functools.partial(
        pl.kernel,
        out_type=jax.ShapeDtypeStruct((NUM_CORES, N_NODES, FDIM), jnp.float32),
        mesh=mesh,
        scratch_types=[
            pltpu.VMEM_SHARED((N_NODES, FDIM), jnp.float32),
            pltpu.VMEM((CHUNK,), jnp.int32),
            pltpu.VMEM((CHUNK,), jnp.int32),
            pltpu.VMEM((CHUNK,), jnp.float32),
            pltpu.VMEM((CHUNK, FDIM), jnp.float32),
            pltpu.VMEM((ZROWS, FDIM), jnp.float32),
            pltpu.SemaphoreType.DMA,
        ],
    )
    def k(support_hbm, src_hbm, dst_hbm, w_hbm, out_hbm,
          acc, srcbuf, dstbuf, wbuf, rowbuf, zbuf, sem):
        c = lax.axis_index("c")
        s = lax.axis_index("s")
        base = (c * NUM_SUBCORES + s) * EDGES_PER_TILE
        row0 = s * ROWS_PER_TILE

        zeros = jnp.zeros((LANES,), jnp.float32)

        def zrow(i, carry):
            for j in range(FDIM // LANES):
                zbuf[i, pl.ds(j * LANES, LANES)] = zeros
            return carry

        lax.fori_loop(0, ZROWS, zrow, 0)
        for z in range(ROWS_PER_TILE // ZROWS):
            pltpu.sync_copy(zbuf, acc.at[pl.ds(row0 + z * ZROWS, ZROWS)])

        @pl.when(s == 0)
        def _zero_tail():
            pltpu.sync_copy(zbuf.at[pl.ds(0, ROWS_EXTRA)],
                            acc.at[pl.ds(ROWS_EXTRA_BASE, ROWS_EXTRA)])

        plsc.subcore_barrier()

        def chunk_body(ci, carry):
            off = base + ci * CHUNK
            pltpu.sync_copy(src_hbm.at[pl.ds(off, CHUNK)], srcbuf)
            g = pltpu.async_copy(support_hbm.at[srcbuf], rowbuf, sem)
            pltpu.sync_copy(dst_hbm.at[pl.ds(off, CHUNK)], dstbuf)
            pltpu.sync_copy(w_hbm.at[pl.ds(off, CHUNK)], wbuf)
            g.wait()

            def scale_group(g, rcarry):
                wv = wbuf[pl.ds(g * LANES, LANES)]
                for r in range(LANES):
                    row = g * LANES + r
                    wr = wv[r]
                    for j in range(FDIM // LANES):
                        sl = pl.ds(j * LANES, LANES)
                        rowbuf[row, sl] = rowbuf[row, sl] * wr
                return rcarry

            lax.fori_loop(0, CHUNK // LANES, scale_group, 0)
            pltpu.sync_copy(rowbuf, acc.at[dstbuf], add=True)
            return carry

        lax.fori_loop(0, N_CHUNKS, chunk_body, 0)
        plsc.subcore_barrier()
        pltpu.sync_copy(acc.at[pl.ds(row0, ROWS_PER_TILE)],
                        out_hbm.at[c, pl.ds(row0, ROWS_PER_TILE)])

        @pl.when(s == 0)
        def _copy_tail():
            pltpu.sync_copy(acc.at[pl.ds(ROWS_EXTRA_BASE, ROWS_EXTRA)],
                            out_hbm.at[c, pl.ds(ROWS_EXTRA_BASE, ROWS_EXTRA)])

    return k(support, src, dst, w)


_BLK = 2000


def _tc_mm(h, W):
    def body(h_ref, w_ref, o_ref):
        o_ref[...] = jnp.dot(h_ref[...], w_ref[...],
                             preferred_element_type=jnp.float32)

    return pl.pallas_call(
        body,
        grid=(N_NODES // _BLK,),
        in_specs=[
            pl.BlockSpec((_BLK, FDIM), lambda i: (i, 0)),
            pl.BlockSpec((FDIM, FDIM), lambda i: (0, 0)),
        ],
        out_specs=pl.BlockSpec((_BLK, FDIM), lambda i: (i, 0)),
        out_shape=jax.ShapeDtypeStruct((N_NODES, FDIM), jnp.float32),
    )(h, W)


def _tc_combine_relu_mm(p, b, W):
    def body(p_ref, b_ref, w_ref, o_ref):
        h = jnp.maximum(p_ref[0] + p_ref[1] + b_ref[...], 0.0)
        o_ref[...] = jnp.dot(h, w_ref[...], preferred_element_type=jnp.float32)

    return pl.pallas_call(
        body,
        grid=(N_NODES // _BLK,),
        in_specs=[
            pl.BlockSpec((NUM_CORES, _BLK, FDIM), lambda i: (0, i, 0)),
            pl.BlockSpec((1, FDIM), lambda i: (0, 0)),
            pl.BlockSpec((FDIM, FDIM), lambda i: (0, 0)),
        ],
        out_specs=pl.BlockSpec((_BLK, FDIM), lambda i: (i, 0)),
        out_shape=jax.ShapeDtypeStruct((N_NODES, FDIM), jnp.float32),
    )(p, b, W)


def _tc_final(p, b3, Wp1, bp1, Wp2, bp2):
    def body(p_ref, b3_ref, wp1_ref, bp1_ref, wp2_ref, bp2_ref,
             z_ref, emb_ref):
        emb = p_ref[0] + p_ref[1] + b3_ref[...]
        emb_ref[...] = emb
        h = jnp.maximum(
            jnp.dot(emb, wp1_ref[...], preferred_element_type=jnp.float32)
            + bp1_ref[...], 0.0)
        z_ref[...] = (
            jnp.dot(h, wp2_ref[...], preferred_element_type=jnp.float32)
            + bp2_ref[...])

    return pl.pallas_call(
        body,
        grid=(N_NODES // _BLK,),
        in_specs=[
            pl.BlockSpec((NUM_CORES, _BLK, FDIM), lambda i: (0, i, 0)),
            pl.BlockSpec((1, FDIM), lambda i: (0, 0)),
            pl.BlockSpec((FDIM, FDIM), lambda i: (0, 0)),
            pl.BlockSpec((1, FDIM), lambda i: (0, 0)),
            pl.BlockSpec((FDIM, FDIM), lambda i: (0, 0)),
            pl.BlockSpec((1, FDIM), lambda i: (0, 0)),
        ],
        out_specs=[
            pl.BlockSpec((_BLK, FDIM), lambda i: (i, 0)),
            pl.BlockSpec((_BLK, FDIM), lambda i: (i, 0)),
        ],
        out_shape=[
            jax.ShapeDtypeStruct((N_NODES, FDIM), jnp.float32),
            jax.ShapeDtypeStruct((N_NODES, FDIM), jnp.float32),
        ],
    )(p, b3, Wp1, bp1, Wp2, bp2)


def kernel(x, edge_index, edge_weight, W1, b1, W2, b2, W3, b3,
           Wp1, bp1, Wp2, bp2):
    src = edge_index[0]
    dst = edge_index[1]
    b1r = b1.reshape(1, FDIM)
    b2r = b2.reshape(1, FDIM)
    b3r = b3.reshape(1, FDIM)
    bp1r = bp1.reshape(1, FDIM)
    bp2r = bp2.reshape(1, FDIM)

    s1 = _tc_mm(x, W1)
    p1 = _sc_aggregate(s1, src, dst, edge_weight)
    s2 = _tc_combine_relu_mm(p1, b1r, W2)
    p2 = _sc_aggregate(s2, src, dst, edge_weight)
    s3 = _tc_combine_relu_mm(p2, b2r, W3)
    p3 = _sc_aggregate(s3, src, dst, edge_weight)
    z, emb = _tc_final(p3, b3r, Wp1, bp1r, Wp2, bp2r)
    return (z, emb)
